# baseline (speedup 1.0000x reference)
import functools

import jax
import jax.numpy as jnp
from jax import lax
from jax.experimental import pallas as pl
from jax.experimental.pallas import tpu as pltpu
from jax.experimental.pallas import tpu_sc as plsc

N = 10000
E = 320000
D = 128
D_EDGE = 16

NC, NS = 2, 16
NW = NC * NS
E_PER = E // NW
CHUNK = 80
NSTEP = E_PER // CHUNK
ZCH = CHUNK
NZ = N // ZCH
ZITER = (NZ + NS - 1) // NS

NBUF_H = 4
NBUF_E = 4


@functools.lru_cache(maxsize=None)
def _mesh():
    return plsc.VectorSubcoreMesh(
        core_axis_name="c", subcore_axis_name="s", num_cores=NC, num_subcores=NS)


def _zero_vmem(buf, rows, width, dtype=jnp.float32):
    if dtype == jnp.float32 or width >= 32:
        lanes = 16 if dtype == jnp.float32 else 32
        zv = jnp.zeros((lanes,), dtype)

        def body(r, _):
            for k in range(width // lanes):
                buf[r, pl.ds(k * lanes, lanes)] = zv
            return 0

        lax.fori_loop(0, rows, body, 0)
    else:
        zv = jnp.zeros((2, 16), dtype)

        def body(r, _):
            buf[pl.ds(2 * r, 2), pl.ds(0, 16)] = zv
            return 0

        lax.fori_loop(0, rows // 2, body, 0)


def _acc_chunks(sub, fn):
    for j in range(ZITER):
        ch = sub + j * NS

        @pl.when(ch < NZ)
        def _():
            fn(ch)


def _ring(nbuf, prologue_idx, gather1, wait_scatter1):
    nround = (NSTEP + nbuf - 1) // nbuf

    def gathers(j):
        for b in range(nbuf):
            ck = j * nbuf + b

            @pl.when(ck < NSTEP)
            def _():
                gather1(ck, b)

    def scatters(j):
        for b in range(nbuf):
            ck = j * nbuf + b

            @pl.when(ck < NSTEP)
            def _():
                wait_scatter1(ck, b)

    for b in range(nbuf):
        prologue_idx(b, b)
    gathers(0)

    def round_body(j, _):
        scatters(j)

        @pl.when(j + 1 < nround)
        def _():
            gathers(j + 1)

        return 0

    lax.fori_loop(0, nround, round_body, 0)


def _sc_hpass_kernel(table, ei, out, sbuf, dbuf, rows, acc,
                     sem_si, sem_di, sem_g):
    c = lax.axis_index("c")
    s = lax.axis_index("s")
    wid = c * NS + s
    ebase = wid * E_PER

    _zero_vmem(rows[0], ZCH, D, jnp.bfloat16)
    _acc_chunks(s, lambda ch: pltpu.sync_copy(rows[0], acc.at[pl.ds(ch * ZCH, ZCH)]))
    plsc.subcore_barrier()

    def idx_start(ck, b):
        off = ebase + ck * CHUNK
        pltpu.async_copy(ei.at[0, pl.ds(off, CHUNK)], sbuf[b], sem_si[b])
        pltpu.async_copy(ei.at[1, pl.ds(off, CHUNK)], dbuf[b], sem_di[b])

    def gather1(ck, b):
        off = ebase + ck * CHUNK
        pltpu.make_async_copy(
            ei.at[0, pl.ds(off, CHUNK)], sbuf[b], sem_si[b]).wait()
        pltpu.async_copy(table.at[sbuf[b]], rows[b], sem_g[b])

    def wait_scatter1(ck, b):
        off = ebase + ck * CHUNK
        pltpu.make_async_copy(table.at[sbuf[b]], rows[b], sem_g[b]).wait()
        pltpu.make_async_copy(
            ei.at[1, pl.ds(off, CHUNK)], dbuf[b], sem_di[b]).wait()
        pltpu.sync_copy(rows[b], acc.at[dbuf[b]], add=True)

        @pl.when(ck + NBUF_H < NSTEP)
        def _():
            idx_start(ck + NBUF_H, b)

    _ring(NBUF_H, idx_start, gather1, wait_scatter1)
    plsc.subcore_barrier()

    def copy_out(ch):
        pltpu.sync_copy(acc.at[pl.ds(ch * ZCH, ZCH)], rows[0])
        pltpu.sync_copy(rows[0], out.at[c, pl.ds(ch * ZCH, ZCH)])

    _acc_chunks(s, copy_out)


@functools.lru_cache(maxsize=None)
def _sc_hpass_call():
    return pl.kernel(
        _sc_hpass_kernel,
        out_type=jax.ShapeDtypeStruct((NC, N, D), jnp.bfloat16),
        mesh=_mesh(),
        scratch_types=[
            tuple(pltpu.VMEM((CHUNK,), jnp.int32) for _ in range(NBUF_H)),
            tuple(pltpu.VMEM((CHUNK,), jnp.int32) for _ in range(NBUF_H)),
            tuple(pltpu.VMEM((CHUNK, D), jnp.bfloat16) for _ in range(NBUF_H)),
            pltpu.VMEM_SHARED((N, D), jnp.bfloat16),
            tuple(pltpu.SemaphoreType.DMA for _ in range(NBUF_H)),
            tuple(pltpu.SemaphoreType.DMA for _ in range(NBUF_H)),
            tuple(pltpu.SemaphoreType.DMA for _ in range(NBUF_H)),
        ],
        compiler_params=pltpu.CompilerParams(use_tc_tiling_on_sc=False),
    )


def _sc_efdeg_kernel(ef, ei, ones_in, tok, out_e, out_d, dbuf, efb, ones,
                     acc_e, acc_d, sem_di, sem_e):
    del tok
    c = lax.axis_index("c")
    s = lax.axis_index("s")
    wid = c * NS + s
    ebase = wid * E_PER

    _zero_vmem(efb[0], ZCH, D_EDGE, jnp.bfloat16)
    _acc_chunks(
        s, lambda ch: pltpu.sync_copy(efb[0], acc_e.at[pl.ds(ch * ZCH, ZCH)]))
    _acc_chunks(
        s, lambda ch: pltpu.sync_copy(efb[0], acc_d.at[pl.ds(ch * ZCH, ZCH)]))
    pltpu.sync_copy(ones_in, ones)
    plsc.subcore_barrier()

    def idx_start(ck, b):
        off = ebase + ck * CHUNK
        pltpu.async_copy(ei.at[1, pl.ds(off, CHUNK)], dbuf[b], sem_di[b])
        pltpu.async_copy(ef.at[pl.ds(off, CHUNK)], efb[b], sem_e[b])

    def gather1(ck, b):
        pass

    def wait_scatter1(ck, b):
        off = ebase + ck * CHUNK
        pltpu.make_async_copy(
            ei.at[1, pl.ds(off, CHUNK)], dbuf[b], sem_di[b]).wait()
        pltpu.make_async_copy(ef.at[pl.ds(off, CHUNK)], efb[b], sem_e[b]).wait()
        pltpu.sync_copy(efb[b], acc_e.at[dbuf[b]], add=True)
        pltpu.sync_copy(ones, acc_d.at[dbuf[b]], add=True)

        @pl.when(ck + NBUF_E < NSTEP)
        def _():
            idx_start(ck + NBUF_E, b)

    _ring(NBUF_E, idx_start, gather1, wait_scatter1)
    plsc.subcore_barrier()

    def copy_out(ch):
        pltpu.sync_copy(acc_e.at[pl.ds(ch * ZCH, ZCH)], efb[0])
        pltpu.sync_copy(efb[0], out_e.at[c, pl.ds(ch * ZCH, ZCH)])
        pltpu.sync_copy(acc_d.at[pl.ds(ch * ZCH, ZCH)], efb[0])
        pltpu.sync_copy(efb[0], out_d.at[c, pl.ds(ch * ZCH, ZCH)])

    _acc_chunks(s, copy_out)


@functools.lru_cache(maxsize=None)
def _sc_efdeg_call():
    return pl.kernel(
        _sc_efdeg_kernel,
        out_type=(jax.ShapeDtypeStruct((NC, N, D_EDGE), jnp.bfloat16),
                  jax.ShapeDtypeStruct((NC, N, D_EDGE), jnp.bfloat16)),
        mesh=_mesh(),
        scratch_types=[
            tuple(pltpu.VMEM((CHUNK,), jnp.int32) for _ in range(NBUF_E)),
            tuple(pltpu.VMEM((CHUNK, D_EDGE), jnp.bfloat16) for _ in range(NBUF_E)),
            pltpu.VMEM((CHUNK, D_EDGE), jnp.bfloat16),
            pltpu.VMEM_SHARED((N, D_EDGE), jnp.bfloat16),
            pltpu.VMEM_SHARED((N, D_EDGE), jnp.bfloat16),
            tuple(pltpu.SemaphoreType.DMA for _ in range(NBUF_E)),
            tuple(pltpu.SemaphoreType.DMA for _ in range(NBUF_E)),
        ],
        compiler_params=pltpu.CompilerParams(use_tc_tiling_on_sc=False),
    )


TC_B = 1000


def _tc0_body(x_ref, w_ref, b_ref, out_ref):
    h = jax.nn.gelu(
        jnp.dot(x_ref[...], w_ref[...], preferred_element_type=jnp.float32)
        + b_ref[...])
    out_ref[...] = h.astype(jnp.bfloat16)


def _agg(s_ref, e_ref, d_ref, wh_ref, we_ref, zb_ref):
    ssum = (s_ref[0].astype(jnp.float32) + s_ref[1].astype(jnp.float32))
    esum = (e_ref[0].astype(jnp.float32) + e_ref[1].astype(jnp.float32))
    dsum = (d_ref[0].astype(jnp.float32) + d_ref[1].astype(jnp.float32))
    return (jnp.dot(ssum, wh_ref[...], preferred_element_type=jnp.float32)
            + jnp.dot(esum, we_ref[...], preferred_element_type=jnp.float32)
            + jnp.dot(dsum, zb_ref[...], preferred_element_type=jnp.float32))


def _tc_mid_body(h_ref, s_ref, e_ref, d_ref, wh_ref, we_ref, zb_ref, out_ref):
    agg = _agg(s_ref, e_ref, d_ref, wh_ref, we_ref, zb_ref)
    out_ref[...] = (h_ref[...].astype(jnp.float32)
                    + jax.nn.gelu(agg)).astype(jnp.bfloat16)


def _tc_fin_body(h_ref, s_ref, e_ref, d_ref, wh_ref, we_ref, zb_ref,
                 w2_ref, b2_ref, out_ref):
    agg = _agg(s_ref, e_ref, d_ref, wh_ref, we_ref, zb_ref)
    h2 = h_ref[...].astype(jnp.float32) + jax.nn.gelu(agg)
    out_ref[...] = (jnp.dot(h2, w2_ref[...], preferred_element_type=jnp.float32)
                    + b2_ref[...])


def _row_spec(b, w):
    return pl.BlockSpec((b, w), lambda i: (i, 0))


def _part_spec(b, w):
    return pl.BlockSpec((NC, b, w), lambda i: (0, i, 0))


def _full_spec(shape):
    return pl.BlockSpec(shape, lambda i: tuple(0 for _ in shape))


def _tc0(x, W1, b1):
    return pl.pallas_call(
        _tc0_body,
        grid=(N // TC_B,),
        in_specs=[_row_spec(TC_B, D), _full_spec((D, D)), _full_spec((1, D))],
        out_specs=_row_spec(TC_B, D),
        out_shape=jax.ShapeDtypeStruct((N, D), jnp.bfloat16),
    )(x, W1, b1)


def _tc_mid(h, sparts, eparts, dparts, wh, we, zb):
    return pl.pallas_call(
        _tc_mid_body,
        grid=(N // TC_B,),
        in_specs=[
            _row_spec(TC_B, D),
            _part_spec(TC_B, D),
            _part_spec(TC_B, D_EDGE),
            _part_spec(TC_B, D_EDGE),
            _full_spec((D, D)),
            _full_spec((D_EDGE, D)),
            _full_spec((D_EDGE, D)),
        ],
        out_specs=_row_spec(TC_B, D),
        out_shape=jax.ShapeDtypeStruct((N, D), jnp.bfloat16),
    )(h, sparts, eparts, dparts, wh, we, zb)


def _tc_fin(h, sparts, eparts, dparts, wh, we, zb, W2, b2):
    return pl.pallas_call(
        _tc_fin_body,
        grid=(N // TC_B,),
        in_specs=[
            _row_spec(TC_B, D),
            _part_spec(TC_B, D),
            _part_spec(TC_B, D_EDGE),
            _part_spec(TC_B, D_EDGE),
            _full_spec((D, D)),
            _full_spec((D_EDGE, D)),
            _full_spec((D_EDGE, D)),
            _full_spec((D, D)),
            _full_spec((1, D)),
        ],
        out_specs=_row_spec(TC_B, D),
        out_shape=jax.ShapeDtypeStruct((N, D), jnp.float32),
    )(h, sparts, eparts, dparts, wh, we, zb, W2, b2)


def _zrow16(bm):
    return jnp.concatenate(
        [bm[None, :], jnp.zeros((D_EDGE - 1, D), jnp.float32)], axis=0)


def kernel(x, edge_index, edge_features, W1, b1, Wm1, bm1, Wm2, bm2, W2, b2):
    ef16 = edge_features.astype(jnp.bfloat16)
    wh1, we1, zb1 = Wm1[:D], Wm1[D:], _zrow16(bm1)
    wh2, we2, zb2 = Wm2[:D], Wm2[D:], _zrow16(bm2)

    h0 = _tc0(x, W1, b1.reshape(1, D))
    s1 = _sc_hpass_call()(h0, edge_index)
    ones_in = jnp.tile(
        (jnp.arange(D_EDGE) < 1).astype(jnp.bfloat16)[None, :], (CHUNK, 1))
    eparts, dparts = _sc_efdeg_call()(ef16, edge_index, ones_in, s1)
    h1 = _tc_mid(h0, s1, eparts, dparts, wh1, we1, zb1)
    s2 = _sc_hpass_call()(h1, edge_index)
    return _tc_fin(h1, s2, eparts, dparts, wh2, we2, zb2, W2, b2.reshape(1, D))


# speedup vs baseline: 1.0196x; 1.0196x over previous
import functools

import jax
import jax.numpy as jnp
from jax import lax
from jax.experimental import pallas as pl
from jax.experimental.pallas import tpu as pltpu
from jax.experimental.pallas import tpu_sc as plsc

N = 10000
E = 320000
D = 128
D_EDGE = 16

NC, NS = 2, 16
NW = NC * NS
E_PER = E // NW
CHUNK = 80
NSTEP = E_PER // CHUNK
ZCH = CHUNK
NZ = N // ZCH
ZITER = (NZ + NS - 1) // NS

NBUF_H = 4
NBUF_E = 4


@functools.lru_cache(maxsize=None)
def _mesh():
    return plsc.VectorSubcoreMesh(
        core_axis_name="c", subcore_axis_name="s", num_cores=NC, num_subcores=NS)


def _zero_vmem(buf, rows, width, dtype=jnp.float32):
    if dtype == jnp.float32 or width >= 32:
        lanes = 16 if dtype == jnp.float32 else 32
        zv = jnp.zeros((lanes,), dtype)

        def body(r, _):
            for k in range(width // lanes):
                buf[r, pl.ds(k * lanes, lanes)] = zv
            return 0

        lax.fori_loop(0, rows, body, 0)
    else:
        zv = jnp.zeros((2, 16), dtype)

        def body(r, _):
            buf[pl.ds(2 * r, 2), pl.ds(0, 16)] = zv
            return 0

        lax.fori_loop(0, rows // 2, body, 0)


def _acc_chunks(sub, fn):
    for j in range(ZITER):
        ch = sub + j * NS

        @pl.when(ch < NZ)
        def _():
            fn(ch)


def _ring(nbuf, prologue_idx, gather1, wait_scatter1):
    nround = (NSTEP + nbuf - 1) // nbuf

    def gathers(j):
        for b in range(nbuf):
            ck = j * nbuf + b

            @pl.when(ck < NSTEP)
            def _():
                gather1(ck, b)

    def scatters(j):
        for b in range(nbuf):
            ck = j * nbuf + b

            @pl.when(ck < NSTEP)
            def _():
                wait_scatter1(ck, b)

    for b in range(nbuf):
        prologue_idx(b, b)
    gathers(0)

    def round_body(j, _):
        scatters(j)

        @pl.when(j + 1 < nround)
        def _():
            gathers(j + 1)

        return 0

    lax.fori_loop(0, nround, round_body, 0)


def _sc_hpass_kernel(table, src, dst, out, sbuf, dbuf, rows, acc,
                     sem_si, sem_di, sem_g):
    c = lax.axis_index("c")
    s = lax.axis_index("s")
    wid = c * NS + s
    ebase = wid * E_PER

    _zero_vmem(rows[0], ZCH, D, jnp.bfloat16)
    _acc_chunks(s, lambda ch: pltpu.sync_copy(rows[0], acc.at[pl.ds(ch * ZCH, ZCH)]))
    plsc.subcore_barrier()

    def idx_start(ck, b):
        off = ebase + ck * CHUNK
        pltpu.async_copy(src.at[pl.ds(off, CHUNK)], sbuf[b], sem_si[b])
        pltpu.async_copy(dst.at[pl.ds(off, CHUNK)], dbuf[b], sem_di[b])

    def gather1(ck, b):
        off = ebase + ck * CHUNK
        pltpu.make_async_copy(src.at[pl.ds(off, CHUNK)], sbuf[b], sem_si[b]).wait()
        pltpu.async_copy(table.at[sbuf[b]], rows[b], sem_g[b])

    def wait_scatter1(ck, b):
        off = ebase + ck * CHUNK
        pltpu.make_async_copy(table.at[sbuf[b]], rows[b], sem_g[b]).wait()
        pltpu.make_async_copy(dst.at[pl.ds(off, CHUNK)], dbuf[b], sem_di[b]).wait()
        pltpu.sync_copy(rows[b], acc.at[dbuf[b]], add=True)

        @pl.when(ck + NBUF_H < NSTEP)
        def _():
            idx_start(ck + NBUF_H, b)

    _ring(NBUF_H, idx_start, gather1, wait_scatter1)
    plsc.subcore_barrier()

    def copy_out(ch):
        pltpu.sync_copy(acc.at[pl.ds(ch * ZCH, ZCH)], rows[0])
        pltpu.sync_copy(rows[0], out.at[c, pl.ds(ch * ZCH, ZCH)])

    _acc_chunks(s, copy_out)


@functools.lru_cache(maxsize=None)
def _sc_hpass_call():
    return pl.kernel(
        _sc_hpass_kernel,
        out_type=jax.ShapeDtypeStruct((NC, N, D), jnp.bfloat16),
        mesh=_mesh(),
        scratch_types=[
            tuple(pltpu.VMEM((CHUNK,), jnp.int32) for _ in range(NBUF_H)),
            tuple(pltpu.VMEM((CHUNK,), jnp.int32) for _ in range(NBUF_H)),
            tuple(pltpu.VMEM((CHUNK, D), jnp.bfloat16) for _ in range(NBUF_H)),
            pltpu.VMEM_SHARED((N, D), jnp.bfloat16),
            tuple(pltpu.SemaphoreType.DMA for _ in range(NBUF_H)),
            tuple(pltpu.SemaphoreType.DMA for _ in range(NBUF_H)),
            tuple(pltpu.SemaphoreType.DMA for _ in range(NBUF_H)),
        ],
        compiler_params=pltpu.CompilerParams(use_tc_tiling_on_sc=False),
    )


def _sc_efdeg_kernel(ef, dst, ones_in, tok, out_e, out_d, dbuf, efb, ones,
                     acc_e, acc_d, sem_di, sem_e):
    del tok
    c = lax.axis_index("c")
    s = lax.axis_index("s")
    wid = c * NS + s
    ebase = wid * E_PER

    _zero_vmem(efb[0], ZCH, D_EDGE, jnp.bfloat16)
    _acc_chunks(
        s, lambda ch: pltpu.sync_copy(efb[0], acc_e.at[pl.ds(ch * ZCH, ZCH)]))
    _acc_chunks(
        s, lambda ch: pltpu.sync_copy(efb[0], acc_d.at[pl.ds(ch * ZCH, ZCH)]))
    pltpu.sync_copy(ones_in, ones)
    plsc.subcore_barrier()

    def idx_start(ck, b):
        off = ebase + ck * CHUNK
        pltpu.async_copy(dst.at[pl.ds(off, CHUNK)], dbuf[b], sem_di[b])
        pltpu.async_copy(ef.at[pl.ds(off, CHUNK)], efb[b], sem_e[b])

    def gather1(ck, b):
        pass

    def wait_scatter1(ck, b):
        off = ebase + ck * CHUNK
        pltpu.make_async_copy(dst.at[pl.ds(off, CHUNK)], dbuf[b], sem_di[b]).wait()
        pltpu.make_async_copy(ef.at[pl.ds(off, CHUNK)], efb[b], sem_e[b]).wait()
        pltpu.sync_copy(efb[b], acc_e.at[dbuf[b]], add=True)
        pltpu.sync_copy(ones, acc_d.at[dbuf[b]], add=True)

        @pl.when(ck + NBUF_E < NSTEP)
        def _():
            idx_start(ck + NBUF_E, b)

    _ring(NBUF_E, idx_start, gather1, wait_scatter1)
    plsc.subcore_barrier()

    def copy_out(ch):
        pltpu.sync_copy(acc_e.at[pl.ds(ch * ZCH, ZCH)], efb[0])
        pltpu.sync_copy(efb[0], out_e.at[c, pl.ds(ch * ZCH, ZCH)])
        pltpu.sync_copy(acc_d.at[pl.ds(ch * ZCH, ZCH)], efb[0])
        pltpu.sync_copy(efb[0], out_d.at[c, pl.ds(ch * ZCH, ZCH)])

    _acc_chunks(s, copy_out)


@functools.lru_cache(maxsize=None)
def _sc_efdeg_call():
    return pl.kernel(
        _sc_efdeg_kernel,
        out_type=(jax.ShapeDtypeStruct((NC, N, D_EDGE), jnp.bfloat16),
                  jax.ShapeDtypeStruct((NC, N, D_EDGE), jnp.bfloat16)),
        mesh=_mesh(),
        scratch_types=[
            tuple(pltpu.VMEM((CHUNK,), jnp.int32) for _ in range(NBUF_E)),
            tuple(pltpu.VMEM((CHUNK, D_EDGE), jnp.bfloat16) for _ in range(NBUF_E)),
            pltpu.VMEM((CHUNK, D_EDGE), jnp.bfloat16),
            pltpu.VMEM_SHARED((N, D_EDGE), jnp.bfloat16),
            pltpu.VMEM_SHARED((N, D_EDGE), jnp.bfloat16),
            tuple(pltpu.SemaphoreType.DMA for _ in range(NBUF_E)),
            tuple(pltpu.SemaphoreType.DMA for _ in range(NBUF_E)),
        ],
        compiler_params=pltpu.CompilerParams(use_tc_tiling_on_sc=False),
    )


TC_B = 1000


def _tc0_body(x_ref, w_ref, b_ref, out_ref):
    h = jax.nn.gelu(
        jnp.dot(x_ref[...], w_ref[...], preferred_element_type=jnp.float32)
        + b_ref[...])
    out_ref[...] = h.astype(jnp.bfloat16)


def _agg(s_ref, e_ref, d_ref, wh_ref, we_ref, zb_ref):
    ssum = (s_ref[0].astype(jnp.float32) + s_ref[1].astype(jnp.float32))
    esum = (e_ref[0].astype(jnp.float32) + e_ref[1].astype(jnp.float32))
    dsum = (d_ref[0].astype(jnp.float32) + d_ref[1].astype(jnp.float32))
    return (jnp.dot(ssum, wh_ref[...], preferred_element_type=jnp.float32)
            + jnp.dot(esum, we_ref[...], preferred_element_type=jnp.float32)
            + jnp.dot(dsum, zb_ref[...], preferred_element_type=jnp.float32))


def _tc_mid_body(h_ref, s_ref, e_ref, d_ref, wh_ref, we_ref, zb_ref, out_ref):
    agg = _agg(s_ref, e_ref, d_ref, wh_ref, we_ref, zb_ref)
    out_ref[...] = (h_ref[...].astype(jnp.float32)
                    + jax.nn.gelu(agg)).astype(jnp.bfloat16)


def _tc_fin_body(h_ref, s_ref, e_ref, d_ref, wh_ref, we_ref, zb_ref,
                 w2_ref, b2_ref, out_ref):
    agg = _agg(s_ref, e_ref, d_ref, wh_ref, we_ref, zb_ref)
    h2 = h_ref[...].astype(jnp.float32) + jax.nn.gelu(agg)
    out_ref[...] = (jnp.dot(h2, w2_ref[...], preferred_element_type=jnp.float32)
                    + b2_ref[...])


def _row_spec(b, w):
    return pl.BlockSpec((b, w), lambda i: (i, 0))


def _part_spec(b, w):
    return pl.BlockSpec((NC, b, w), lambda i: (0, i, 0))


def _full_spec(shape):
    return pl.BlockSpec(shape, lambda i: tuple(0 for _ in shape))


def _tc0(x, W1, b1):
    return pl.pallas_call(
        _tc0_body,
        grid=(N // TC_B,),
        in_specs=[_row_spec(TC_B, D), _full_spec((D, D)), _full_spec((1, D))],
        out_specs=_row_spec(TC_B, D),
        out_shape=jax.ShapeDtypeStruct((N, D), jnp.bfloat16),
    )(x, W1, b1)


def _tc_mid(h, sparts, eparts, dparts, wh, we, zb):
    return pl.pallas_call(
        _tc_mid_body,
        grid=(N // TC_B,),
        in_specs=[
            _row_spec(TC_B, D),
            _part_spec(TC_B, D),
            _part_spec(TC_B, D_EDGE),
            _part_spec(TC_B, D_EDGE),
            _full_spec((D, D)),
            _full_spec((D_EDGE, D)),
            _full_spec((D_EDGE, D)),
        ],
        out_specs=_row_spec(TC_B, D),
        out_shape=jax.ShapeDtypeStruct((N, D), jnp.bfloat16),
    )(h, sparts, eparts, dparts, wh, we, zb)


def _tc_fin(h, sparts, eparts, dparts, wh, we, zb, W2, b2):
    return pl.pallas_call(
        _tc_fin_body,
        grid=(N // TC_B,),
        in_specs=[
            _row_spec(TC_B, D),
            _part_spec(TC_B, D),
            _part_spec(TC_B, D_EDGE),
            _part_spec(TC_B, D_EDGE),
            _full_spec((D, D)),
            _full_spec((D_EDGE, D)),
            _full_spec((D_EDGE, D)),
            _full_spec((D, D)),
            _full_spec((1, D)),
        ],
        out_specs=_row_spec(TC_B, D),
        out_shape=jax.ShapeDtypeStruct((N, D), jnp.float32),
    )(h, sparts, eparts, dparts, wh, we, zb, W2, b2)


def _zrow16(bm):
    return jnp.concatenate(
        [bm[None, :], jnp.zeros((D_EDGE - 1, D), jnp.float32)], axis=0)


def kernel(x, edge_index, edge_features, W1, b1, Wm1, bm1, Wm2, bm2, W2, b2):
    ef16 = edge_features.astype(jnp.bfloat16)
    wh1, we1, zb1 = Wm1[:D], Wm1[D:], _zrow16(bm1)
    wh2, we2, zb2 = Wm2[:D], Wm2[D:], _zrow16(bm2)

    src = edge_index[0]
    dst = edge_index[1]
    h0 = _tc0(x, W1, b1.reshape(1, D))
    s1 = _sc_hpass_call()(h0, src, dst)
    ones_in = jnp.tile(
        (jnp.arange(D_EDGE) < 1).astype(jnp.bfloat16)[None, :], (CHUNK, 1))
    eparts, dparts = _sc_efdeg_call()(ef16, dst, ones_in, s1)
    h1 = _tc_mid(h0, s1, eparts, dparts, wh1, we1, zb1)
    s2 = _sc_hpass_call()(h1, src, dst)
    return _tc_fin(h1, s2, eparts, dparts, wh2, we2, zb2, W2, b2.reshape(1, D))


# speedup vs baseline: 1.2544x; 1.2302x over previous
import functools

import jax
import jax.numpy as jnp
from jax import lax
from jax.experimental import pallas as pl
from jax.experimental.pallas import tpu as pltpu
from jax.experimental.pallas import tpu_sc as plsc

N = 10000
E = 320000
D = 128
D_EDGE = 16

NC, NS = 2, 16
NW = NC * NS
E_PER = E // NW
CHUNK = 80
NSTEP = E_PER // CHUNK
ZCH = CHUNK
NZ = N // ZCH
ZITER = (NZ + NS - 1) // NS

NBUF_H = 4
NBUF_E = 4


@functools.lru_cache(maxsize=None)
def _mesh():
    return plsc.VectorSubcoreMesh(
        core_axis_name="c", subcore_axis_name="s", num_cores=NC, num_subcores=NS)


def _zero_vmem(buf, rows, width, dtype=jnp.float32):
    if dtype == jnp.float32 or width >= 32:
        lanes = 16 if dtype == jnp.float32 else 32
        zv = jnp.zeros((lanes,), dtype)

        def body(r, _):
            for k in range(width // lanes):
                buf[r, pl.ds(k * lanes, lanes)] = zv
            return 0

        lax.fori_loop(0, rows, body, 0)
    else:
        zv = jnp.zeros((2, 16), dtype)

        def body(r, _):
            buf[pl.ds(2 * r, 2), pl.ds(0, 16)] = zv
            return 0

        lax.fori_loop(0, rows // 2, body, 0)


def _acc_chunks(sub, fn):
    for j in range(ZITER):
        ch = sub + j * NS

        @pl.when(ch < NZ)
        def _():
            fn(ch)


def _ring(nbuf, prologue_idx, gather1, wait_scatter1):
    nround = (NSTEP + nbuf - 1) // nbuf

    def gathers(j):
        for b in range(nbuf):
            ck = j * nbuf + b

            @pl.when(ck < NSTEP)
            def _():
                gather1(ck, b)

    def scatters(j):
        for b in range(nbuf):
            ck = j * nbuf + b

            @pl.when(ck < NSTEP)
            def _():
                wait_scatter1(ck, b)

    for b in range(nbuf):
        prologue_idx(b, b)
    gathers(0)

    def round_body(j, _):
        scatters(j)

        @pl.when(j + 1 < nround)
        def _():
            gathers(j + 1)

        return 0

    lax.fori_loop(0, nround, round_body, 0)


def _sc_hpass_kernel(table, src, dst, out, sbuf, dbuf, rows, acc,
                     sem_si, sem_di, sem_g):
    c = lax.axis_index("c")
    s = lax.axis_index("s")
    wid = c * NS + s
    ebase = wid * E_PER

    _zero_vmem(rows[0], ZCH, D, jnp.bfloat16)
    _acc_chunks(s, lambda ch: pltpu.sync_copy(rows[0], acc.at[pl.ds(ch * ZCH, ZCH)]))
    plsc.subcore_barrier()

    def idx_start(ck, b):
        off = ebase + ck * CHUNK
        pltpu.async_copy(src.at[pl.ds(off, CHUNK)], sbuf[b], sem_si[b])
        pltpu.async_copy(dst.at[pl.ds(off, CHUNK)], dbuf[b], sem_di[b])

    def gather1(ck, b):
        off = ebase + ck * CHUNK
        pltpu.make_async_copy(src.at[pl.ds(off, CHUNK)], sbuf[b], sem_si[b]).wait()
        pltpu.async_copy(table.at[sbuf[b]], rows[b], sem_g[b])

    def wait_scatter1(ck, b):
        off = ebase + ck * CHUNK
        pltpu.make_async_copy(table.at[sbuf[b]], rows[b], sem_g[b]).wait()
        pltpu.make_async_copy(dst.at[pl.ds(off, CHUNK)], dbuf[b], sem_di[b]).wait()
        pltpu.sync_copy(rows[b], acc.at[dbuf[b]], add=True)

        @pl.when(ck + NBUF_H < NSTEP)
        def _():
            idx_start(ck + NBUF_H, b)

    _ring(NBUF_H, idx_start, gather1, wait_scatter1)
    plsc.subcore_barrier()

    def copy_out(ch):
        pltpu.sync_copy(acc.at[pl.ds(ch * ZCH, ZCH)], rows[0])
        pltpu.sync_copy(rows[0], out.at[c, pl.ds(ch * ZCH, ZCH)])

    _acc_chunks(s, copy_out)


@functools.lru_cache(maxsize=None)
def _sc_hpass_call():
    return pl.kernel(
        _sc_hpass_kernel,
        out_type=jax.ShapeDtypeStruct((NC, N, D), jnp.bfloat16),
        mesh=_mesh(),
        scratch_types=[
            tuple(pltpu.VMEM((CHUNK,), jnp.int32) for _ in range(NBUF_H)),
            tuple(pltpu.VMEM((CHUNK,), jnp.int32) for _ in range(NBUF_H)),
            tuple(pltpu.VMEM((CHUNK, D), jnp.bfloat16) for _ in range(NBUF_H)),
            pltpu.VMEM_SHARED((N, D), jnp.bfloat16),
            tuple(pltpu.SemaphoreType.DMA for _ in range(NBUF_H)),
            tuple(pltpu.SemaphoreType.DMA for _ in range(NBUF_H)),
            tuple(pltpu.SemaphoreType.DMA for _ in range(NBUF_H)),
        ],
        compiler_params=pltpu.CompilerParams(use_tc_tiling_on_sc=False),
    )


def _sc_efdeg_kernel(ef, dst, ones_in, tok, out_e, out_d, dbuf, efb, ones,
                     acc_e, acc_d, sem_di, sem_e):
    del tok
    c = lax.axis_index("c")
    s = lax.axis_index("s")
    wid = c * NS + s
    ebase = wid * E_PER

    _zero_vmem(efb[0], ZCH, D_EDGE)
    _acc_chunks(
        s, lambda ch: pltpu.sync_copy(efb[0], acc_e.at[pl.ds(ch * ZCH, ZCH)]))
    _acc_chunks(
        s, lambda ch: pltpu.sync_copy(efb[0], acc_d.at[pl.ds(ch * ZCH, ZCH)]))
    pltpu.sync_copy(ones_in, ones)
    plsc.subcore_barrier()

    def idx_start(ck, b):
        off = ebase + ck * CHUNK
        pltpu.async_copy(dst.at[pl.ds(off, CHUNK)], dbuf[b], sem_di[b])
        pltpu.async_copy(ef.at[pl.ds(off, CHUNK)], efb[b], sem_e[b])

    def gather1(ck, b):
        pass

    def wait_scatter1(ck, b):
        off = ebase + ck * CHUNK
        pltpu.make_async_copy(dst.at[pl.ds(off, CHUNK)], dbuf[b], sem_di[b]).wait()
        pltpu.make_async_copy(ef.at[pl.ds(off, CHUNK)], efb[b], sem_e[b]).wait()
        pltpu.sync_copy(efb[b], acc_e.at[dbuf[b]], add=True)
        pltpu.sync_copy(ones, acc_d.at[dbuf[b]], add=True)

        @pl.when(ck + NBUF_E < NSTEP)
        def _():
            idx_start(ck + NBUF_E, b)

    _ring(NBUF_E, idx_start, gather1, wait_scatter1)
    plsc.subcore_barrier()

    def copy_out(ch):
        pltpu.sync_copy(acc_e.at[pl.ds(ch * ZCH, ZCH)], efb[0])
        pltpu.sync_copy(efb[0], out_e.at[c, pl.ds(ch * ZCH, ZCH)])
        pltpu.sync_copy(acc_d.at[pl.ds(ch * ZCH, ZCH)], efb[0])
        pltpu.sync_copy(efb[0], out_d.at[c, pl.ds(ch * ZCH, ZCH)])

    _acc_chunks(s, copy_out)


@functools.lru_cache(maxsize=None)
def _sc_efdeg_call():
    return pl.kernel(
        _sc_efdeg_kernel,
        out_type=(jax.ShapeDtypeStruct((NC, N, D_EDGE), jnp.float32),
                  jax.ShapeDtypeStruct((NC, N, D_EDGE), jnp.float32)),
        mesh=_mesh(),
        scratch_types=[
            tuple(pltpu.VMEM((CHUNK,), jnp.int32) for _ in range(NBUF_E)),
            tuple(pltpu.VMEM((CHUNK, D_EDGE), jnp.float32) for _ in range(NBUF_E)),
            pltpu.VMEM((CHUNK, D_EDGE), jnp.float32),
            pltpu.VMEM_SHARED((N, D_EDGE), jnp.float32),
            pltpu.VMEM_SHARED((N, D_EDGE), jnp.float32),
            tuple(pltpu.SemaphoreType.DMA for _ in range(NBUF_E)),
            tuple(pltpu.SemaphoreType.DMA for _ in range(NBUF_E)),
        ],
        compiler_params=pltpu.CompilerParams(use_tc_tiling_on_sc=False),
    )


TC_B = 1000


def _tc0_body(x_ref, w_ref, b_ref, out_ref):
    h = jax.nn.gelu(
        jnp.dot(x_ref[...], w_ref[...], preferred_element_type=jnp.float32)
        + b_ref[...])
    out_ref[...] = h.astype(jnp.bfloat16)


def _agg(s_ref, e_ref, d_ref, wh_ref, we_ref, zb_ref):
    ssum = (s_ref[0].astype(jnp.float32) + s_ref[1].astype(jnp.float32))
    esum = (e_ref[0].astype(jnp.float32) + e_ref[1].astype(jnp.float32))
    dsum = (d_ref[0].astype(jnp.float32) + d_ref[1].astype(jnp.float32))
    return (jnp.dot(ssum, wh_ref[...], preferred_element_type=jnp.float32)
            + jnp.dot(esum, we_ref[...], preferred_element_type=jnp.float32)
            + jnp.dot(dsum, zb_ref[...], preferred_element_type=jnp.float32))


def _tc_mid_body(h_ref, s_ref, e_ref, d_ref, wh_ref, we_ref, zb_ref, out_ref):
    agg = _agg(s_ref, e_ref, d_ref, wh_ref, we_ref, zb_ref)
    out_ref[...] = (h_ref[...].astype(jnp.float32)
                    + jax.nn.gelu(agg)).astype(jnp.bfloat16)


def _tc_fin_body(h_ref, s_ref, e_ref, d_ref, wh_ref, we_ref, zb_ref,
                 w2_ref, b2_ref, out_ref):
    agg = _agg(s_ref, e_ref, d_ref, wh_ref, we_ref, zb_ref)
    h2 = h_ref[...].astype(jnp.float32) + jax.nn.gelu(agg)
    out_ref[...] = (jnp.dot(h2, w2_ref[...], preferred_element_type=jnp.float32)
                    + b2_ref[...])


def _row_spec(b, w):
    return pl.BlockSpec((b, w), lambda i: (i, 0))


def _part_spec(b, w):
    return pl.BlockSpec((NC, b, w), lambda i: (0, i, 0))


def _full_spec(shape):
    return pl.BlockSpec(shape, lambda i: tuple(0 for _ in shape))


def _tc0(x, W1, b1):
    return pl.pallas_call(
        _tc0_body,
        grid=(N // TC_B,),
        in_specs=[_row_spec(TC_B, D), _full_spec((D, D)), _full_spec((1, D))],
        out_specs=_row_spec(TC_B, D),
        out_shape=jax.ShapeDtypeStruct((N, D), jnp.bfloat16),
    )(x, W1, b1)


def _tc_mid(h, sparts, eparts, dparts, wh, we, zb):
    return pl.pallas_call(
        _tc_mid_body,
        grid=(N // TC_B,),
        in_specs=[
            _row_spec(TC_B, D),
            _part_spec(TC_B, D),
            _part_spec(TC_B, D_EDGE),
            _part_spec(TC_B, D_EDGE),
            _full_spec((D, D)),
            _full_spec((D_EDGE, D)),
            _full_spec((D_EDGE, D)),
        ],
        out_specs=_row_spec(TC_B, D),
        out_shape=jax.ShapeDtypeStruct((N, D), jnp.bfloat16),
    )(h, sparts, eparts, dparts, wh, we, zb)


def _tc_fin(h, sparts, eparts, dparts, wh, we, zb, W2, b2):
    return pl.pallas_call(
        _tc_fin_body,
        grid=(N // TC_B,),
        in_specs=[
            _row_spec(TC_B, D),
            _part_spec(TC_B, D),
            _part_spec(TC_B, D_EDGE),
            _part_spec(TC_B, D_EDGE),
            _full_spec((D, D)),
            _full_spec((D_EDGE, D)),
            _full_spec((D_EDGE, D)),
            _full_spec((D, D)),
            _full_spec((1, D)),
        ],
        out_specs=_row_spec(TC_B, D),
        out_shape=jax.ShapeDtypeStruct((N, D), jnp.float32),
    )(h, sparts, eparts, dparts, wh, we, zb, W2, b2)


def _zrow16(bm):
    return jnp.concatenate(
        [bm[None, :], jnp.zeros((D_EDGE - 1, D), jnp.float32)], axis=0)


def kernel(x, edge_index, edge_features, W1, b1, Wm1, bm1, Wm2, bm2, W2, b2):
    wh1, we1, zb1 = Wm1[:D], Wm1[D:], _zrow16(bm1)
    wh2, we2, zb2 = Wm2[:D], Wm2[D:], _zrow16(bm2)

    src = edge_index[0]
    dst = edge_index[1]
    h0 = _tc0(x, W1, b1.reshape(1, D))
    s1 = _sc_hpass_call()(h0, src, dst)
    ones_in = jnp.tile(
        (jnp.arange(D_EDGE) < 1).astype(jnp.float32)[None, :], (CHUNK, 1))
    eparts, dparts = _sc_efdeg_call()(edge_features, dst, ones_in, s1)
    h1 = _tc_mid(h0, s1, eparts, dparts, wh1, we1, zb1)
    s2 = _sc_hpass_call()(h1, src, dst)
    return _tc_fin(h1, s2, eparts, dparts, wh2, we2, zb2, W2, b2.reshape(1, D))


# speedup vs baseline: 1.2651x; 1.0086x over previous
import functools

import jax
import jax.numpy as jnp
from jax import lax
from jax.experimental import pallas as pl
from jax.experimental.pallas import tpu as pltpu
from jax.experimental.pallas import tpu_sc as plsc

N = 10000
E = 320000
D = 128
D_EDGE = 16

NC, NS = 2, 16
NW = NC * NS
E_PER = E // NW
CHUNK = 80
NSTEP = E_PER // CHUNK
ZCH = CHUNK
NZ = N // ZCH
ZITER = (NZ + NS - 1) // NS

NBUF_H = 8
NBUF_E = 4


@functools.lru_cache(maxsize=None)
def _mesh():
    return plsc.VectorSubcoreMesh(
        core_axis_name="c", subcore_axis_name="s", num_cores=NC, num_subcores=NS)


def _zero_vmem(buf, rows, width, dtype=jnp.float32):
    if dtype == jnp.float32 or width >= 32:
        lanes = 16 if dtype == jnp.float32 else 32
        zv = jnp.zeros((lanes,), dtype)

        def body(r, _):
            for k in range(width // lanes):
                buf[r, pl.ds(k * lanes, lanes)] = zv
            return 0

        lax.fori_loop(0, rows, body, 0)
    else:
        zv = jnp.zeros((2, 16), dtype)

        def body(r, _):
            buf[pl.ds(2 * r, 2), pl.ds(0, 16)] = zv
            return 0

        lax.fori_loop(0, rows // 2, body, 0)


def _acc_chunks(sub, fn):
    for j in range(ZITER):
        ch = sub + j * NS

        @pl.when(ch < NZ)
        def _():
            fn(ch)


def _ring(nbuf, prologue_idx, gather1, wait_scatter1):
    nround = (NSTEP + nbuf - 1) // nbuf

    def gathers(j):
        for b in range(nbuf):
            ck = j * nbuf + b

            @pl.when(ck < NSTEP)
            def _():
                gather1(ck, b)

    def scatters(j):
        for b in range(nbuf):
            ck = j * nbuf + b

            @pl.when(ck < NSTEP)
            def _():
                wait_scatter1(ck, b)

    for b in range(nbuf):
        prologue_idx(b, b)
    gathers(0)

    def round_body(j, _):
        scatters(j)

        @pl.when(j + 1 < nround)
        def _():
            gathers(j + 1)

        return 0

    lax.fori_loop(0, nround, round_body, 0)


def _sc_hpass_kernel(table, src, dst, out, sbuf, dbuf, rows, acc,
                     sem_si, sem_di, sem_g):
    c = lax.axis_index("c")
    s = lax.axis_index("s")
    wid = c * NS + s
    ebase = wid * E_PER

    _zero_vmem(rows[0], ZCH, D, jnp.bfloat16)
    _acc_chunks(s, lambda ch: pltpu.sync_copy(rows[0], acc.at[pl.ds(ch * ZCH, ZCH)]))
    plsc.subcore_barrier()

    def idx_start(ck, b):
        off = ebase + ck * CHUNK
        pltpu.async_copy(src.at[pl.ds(off, CHUNK)], sbuf[b], sem_si[b])
        pltpu.async_copy(dst.at[pl.ds(off, CHUNK)], dbuf[b], sem_di[b])

    def gather1(ck, b):
        off = ebase + ck * CHUNK
        pltpu.make_async_copy(src.at[pl.ds(off, CHUNK)], sbuf[b], sem_si[b]).wait()
        pltpu.async_copy(table.at[sbuf[b]], rows[b], sem_g[b])

    def wait_scatter1(ck, b):
        off = ebase + ck * CHUNK
        pltpu.make_async_copy(table.at[sbuf[b]], rows[b], sem_g[b]).wait()
        pltpu.make_async_copy(dst.at[pl.ds(off, CHUNK)], dbuf[b], sem_di[b]).wait()
        pltpu.sync_copy(rows[b], acc.at[dbuf[b]], add=True)

        @pl.when(ck + NBUF_H < NSTEP)
        def _():
            idx_start(ck + NBUF_H, b)

    _ring(NBUF_H, idx_start, gather1, wait_scatter1)
    plsc.subcore_barrier()

    def copy_out(ch):
        pltpu.sync_copy(acc.at[pl.ds(ch * ZCH, ZCH)], rows[0])
        pltpu.sync_copy(rows[0], out.at[c, pl.ds(ch * ZCH, ZCH)])

    _acc_chunks(s, copy_out)


@functools.lru_cache(maxsize=None)
def _sc_hpass_call():
    return pl.kernel(
        _sc_hpass_kernel,
        out_type=jax.ShapeDtypeStruct((NC, N, D), jnp.bfloat16),
        mesh=_mesh(),
        scratch_types=[
            tuple(pltpu.VMEM((CHUNK,), jnp.int32) for _ in range(NBUF_H)),
            tuple(pltpu.VMEM((CHUNK,), jnp.int32) for _ in range(NBUF_H)),
            tuple(pltpu.VMEM((CHUNK, D), jnp.bfloat16) for _ in range(NBUF_H)),
            pltpu.VMEM_SHARED((N, D), jnp.bfloat16),
            tuple(pltpu.SemaphoreType.DMA for _ in range(NBUF_H)),
            tuple(pltpu.SemaphoreType.DMA for _ in range(NBUF_H)),
            tuple(pltpu.SemaphoreType.DMA for _ in range(NBUF_H)),
        ],
        compiler_params=pltpu.CompilerParams(use_tc_tiling_on_sc=False),
    )


def _sc_efdeg_kernel(ef, dst, tok, out_e, out_d, dbuf, efb, ones,
                     acc_e, acc_d, sem_di, sem_e):
    del tok
    c = lax.axis_index("c")
    s = lax.axis_index("s")
    wid = c * NS + s
    ebase = wid * E_PER

    _zero_vmem(efb[0], ZCH, D_EDGE)
    _acc_chunks(
        s, lambda ch: pltpu.sync_copy(efb[0], acc_e.at[pl.ds(ch * ZCH, ZCH)]))
    _acc_chunks(
        s, lambda ch: pltpu.sync_copy(efb[0], acc_d.at[pl.ds(ch * ZCH, ZCH)]))
    onerow = jnp.where(lax.iota(jnp.int32, 16) == 0, 1.0, 0.0).astype(jnp.float32)

    def fill(r, _):
        ones[r, pl.ds(0, 16)] = onerow
        return 0

    lax.fori_loop(0, CHUNK, fill, 0)
    plsc.subcore_barrier()

    def idx_start(ck, b):
        off = ebase + ck * CHUNK
        pltpu.async_copy(dst.at[pl.ds(off, CHUNK)], dbuf[b], sem_di[b])
        pltpu.async_copy(ef.at[pl.ds(off, CHUNK)], efb[b], sem_e[b])

    def gather1(ck, b):
        pass

    def wait_scatter1(ck, b):
        off = ebase + ck * CHUNK
        pltpu.make_async_copy(dst.at[pl.ds(off, CHUNK)], dbuf[b], sem_di[b]).wait()
        pltpu.make_async_copy(ef.at[pl.ds(off, CHUNK)], efb[b], sem_e[b]).wait()
        pltpu.sync_copy(efb[b], acc_e.at[dbuf[b]], add=True)
        pltpu.sync_copy(ones, acc_d.at[dbuf[b]], add=True)

        @pl.when(ck + NBUF_E < NSTEP)
        def _():
            idx_start(ck + NBUF_E, b)

    _ring(NBUF_E, idx_start, gather1, wait_scatter1)
    plsc.subcore_barrier()

    def copy_out(ch):
        pltpu.sync_copy(acc_e.at[pl.ds(ch * ZCH, ZCH)], efb[0])
        pltpu.sync_copy(efb[0], out_e.at[c, pl.ds(ch * ZCH, ZCH)])
        pltpu.sync_copy(acc_d.at[pl.ds(ch * ZCH, ZCH)], efb[0])
        pltpu.sync_copy(efb[0], out_d.at[c, pl.ds(ch * ZCH, ZCH)])

    _acc_chunks(s, copy_out)


@functools.lru_cache(maxsize=None)
def _sc_efdeg_call():
    return pl.kernel(
        _sc_efdeg_kernel,
        out_type=(jax.ShapeDtypeStruct((NC, N, D_EDGE), jnp.float32),
                  jax.ShapeDtypeStruct((NC, N, D_EDGE), jnp.float32)),
        mesh=_mesh(),
        scratch_types=[
            tuple(pltpu.VMEM((CHUNK,), jnp.int32) for _ in range(NBUF_E)),
            tuple(pltpu.VMEM((CHUNK, D_EDGE), jnp.float32) for _ in range(NBUF_E)),
            pltpu.VMEM((CHUNK, D_EDGE), jnp.float32),
            pltpu.VMEM_SHARED((N, D_EDGE), jnp.float32),
            pltpu.VMEM_SHARED((N, D_EDGE), jnp.float32),
            tuple(pltpu.SemaphoreType.DMA for _ in range(NBUF_E)),
            tuple(pltpu.SemaphoreType.DMA for _ in range(NBUF_E)),
        ],
        compiler_params=pltpu.CompilerParams(use_tc_tiling_on_sc=False),
    )


TC_B = 1000


def _tc0_body(x_ref, w_ref, b_ref, out_ref):
    h = jax.nn.gelu(
        jnp.dot(x_ref[...], w_ref[...], preferred_element_type=jnp.float32)
        + b_ref[...])
    out_ref[...] = h.astype(jnp.bfloat16)


def _agg(s_ref, e_ref, d_ref, wh_ref, we_ref, zb_ref):
    ssum = (s_ref[0].astype(jnp.float32) + s_ref[1].astype(jnp.float32))
    esum = (e_ref[0].astype(jnp.float32) + e_ref[1].astype(jnp.float32))
    dsum = (d_ref[0].astype(jnp.float32) + d_ref[1].astype(jnp.float32))
    return (jnp.dot(ssum, wh_ref[...], preferred_element_type=jnp.float32)
            + jnp.dot(esum, we_ref[...], preferred_element_type=jnp.float32)
            + jnp.dot(dsum, zb_ref[...], preferred_element_type=jnp.float32))


def _tc_mid_body(h_ref, s_ref, e_ref, d_ref, wh_ref, we_ref, zb_ref, out_ref):
    agg = _agg(s_ref, e_ref, d_ref, wh_ref, we_ref, zb_ref)
    out_ref[...] = (h_ref[...].astype(jnp.float32)
                    + jax.nn.gelu(agg)).astype(jnp.bfloat16)


def _tc_fin_body(h_ref, s_ref, e_ref, d_ref, wh_ref, we_ref, zb_ref,
                 w2_ref, b2_ref, out_ref):
    agg = _agg(s_ref, e_ref, d_ref, wh_ref, we_ref, zb_ref)
    h2 = h_ref[...].astype(jnp.float32) + jax.nn.gelu(agg)
    out_ref[...] = (jnp.dot(h2, w2_ref[...], preferred_element_type=jnp.float32)
                    + b2_ref[...])


def _row_spec(b, w):
    return pl.BlockSpec((b, w), lambda i: (i, 0))


def _part_spec(b, w):
    return pl.BlockSpec((NC, b, w), lambda i: (0, i, 0))


def _full_spec(shape):
    return pl.BlockSpec(shape, lambda i: tuple(0 for _ in shape))


def _tc0(x, W1, b1):
    return pl.pallas_call(
        _tc0_body,
        grid=(N // TC_B,),
        in_specs=[_row_spec(TC_B, D), _full_spec((D, D)), _full_spec((1, D))],
        out_specs=_row_spec(TC_B, D),
        out_shape=jax.ShapeDtypeStruct((N, D), jnp.bfloat16),
    )(x, W1, b1)


def _tc_mid(h, sparts, eparts, dparts, wh, we, zb):
    return pl.pallas_call(
        _tc_mid_body,
        grid=(N // TC_B,),
        in_specs=[
            _row_spec(TC_B, D),
            _part_spec(TC_B, D),
            _part_spec(TC_B, D_EDGE),
            _part_spec(TC_B, D_EDGE),
            _full_spec((D, D)),
            _full_spec((D_EDGE, D)),
            _full_spec((D_EDGE, D)),
        ],
        out_specs=_row_spec(TC_B, D),
        out_shape=jax.ShapeDtypeStruct((N, D), jnp.bfloat16),
    )(h, sparts, eparts, dparts, wh, we, zb)


def _tc_fin(h, sparts, eparts, dparts, wh, we, zb, W2, b2):
    return pl.pallas_call(
        _tc_fin_body,
        grid=(N // TC_B,),
        in_specs=[
            _row_spec(TC_B, D),
            _part_spec(TC_B, D),
            _part_spec(TC_B, D_EDGE),
            _part_spec(TC_B, D_EDGE),
            _full_spec((D, D)),
            _full_spec((D_EDGE, D)),
            _full_spec((D_EDGE, D)),
            _full_spec((D, D)),
            _full_spec((1, D)),
        ],
        out_specs=_row_spec(TC_B, D),
        out_shape=jax.ShapeDtypeStruct((N, D), jnp.float32),
    )(h, sparts, eparts, dparts, wh, we, zb, W2, b2)


def _zrow16(bm):
    return jnp.concatenate(
        [bm[None, :], jnp.zeros((D_EDGE - 1, D), jnp.float32)], axis=0)


def kernel(x, edge_index, edge_features, W1, b1, Wm1, bm1, Wm2, bm2, W2, b2):
    wh1, we1, zb1 = Wm1[:D], Wm1[D:], _zrow16(bm1)
    wh2, we2, zb2 = Wm2[:D], Wm2[D:], _zrow16(bm2)

    src = edge_index[0]
    dst = edge_index[1]
    h0 = _tc0(x, W1, b1.reshape(1, D))
    s1 = _sc_hpass_call()(h0, src, dst)
    eparts, dparts = _sc_efdeg_call()(edge_features, dst, s1)
    h1 = _tc_mid(h0, s1, eparts, dparts, wh1, we1, zb1)
    s2 = _sc_hpass_call()(h1, src, dst)
    return _tc_fin(h1, s2, eparts, dparts, wh2, we2, zb2, W2, b2.reshape(1, D))
